# Initial kernel scaffold; baseline (speedup 1.0000x reference)
import functools

import jax
import jax.numpy as jnp
from jax import lax
from jax.experimental import pallas as pl
from jax.experimental.pallas import tpu as pltpu

_NUM_HIDDENS = 128
_NUM_EMBEDDINGS = 8192
_EMBEDDING_DIM = 64
_COMMITMENT_COST = 0.25

_TB = 1024
_KB = 1024


def _cpad(x, p):
    return jnp.concatenate([x[:, :, -p:], x, x[:, :, :p]], axis=2)


def _conv1d(x, W, b=None, stride=1, pad=0, circular=False):
    if circular and pad > 0:
        x = _cpad(x, pad)
        padding = ((0, 0),)
    else:
        padding = ((pad, pad),)
    out = lax.conv_general_dilated(x, W, (stride,), padding,
                                   dimension_numbers=('NCH', 'OIH', 'NCH'))
    if b is not None:
        out = out + b[None, :, None]
    return out


def _conv_transpose1d(x, W, b, stride, pad):
    k = W.shape[-1]
    out = lax.conv_general_dilated(x, W[:, :, ::-1], (1,),
                                   ((k - 1 - pad, k - 1 - pad),),
                                   lhs_dilation=(stride,),
                                   dimension_numbers=('NCH', 'OIH', 'NCH'))
    return out + b[None, :, None]


def _residual_stack(x, ws):
    for (w1, w2) in ws:
        h = jax.nn.relu(x)
        h = _conv1d(h, w1, stride=1, pad=1, circular=True)
        h = jax.nn.relu(h)
        h = _conv1d(h, w2)
        x = x + h
    return jax.nn.relu(x)


def _vq_tc_kernel(flat_ref, e_ref, e2_ref, idx_ref, lat_ref,
                  min_ref, arg_ref):
    j = pl.program_id(1)
    nk = pl.num_programs(1)
    i = pl.program_id(0)

    flat = flat_ref[...]
    e = e_ref[...]
    s = lax.dot_general(flat, e, (((1,), (1,)), ((), ())),
                        preferred_element_type=jnp.float32,
                        precision=lax.Precision.HIGHEST)
    s = e2_ref[...] - 2.0 * s

    m = jnp.min(s, axis=1, keepdims=True)
    col = lax.broadcasted_iota(jnp.int32, s.shape, 1)
    arg = jnp.min(jnp.where(s == m, col, _NUM_EMBEDDINGS), axis=1,
                  keepdims=True) + j * _KB

    @pl.when(j == 0)
    def _init():
        min_ref[...] = m
        arg_ref[...] = arg

    @pl.when(j > 0)
    def _update():
        better = m < min_ref[...]
        arg_ref[...] = jnp.where(better, arg, arg_ref[...])
        min_ref[...] = jnp.where(better, m, min_ref[...])

    @pl.when(j == nk - 1)
    def _finish():
        idx_ref[...] = arg_ref[...]
        z2 = jnp.sum(flat * flat, axis=1, keepdims=True)
        part = jnp.sum(min_ref[...] + z2)

        @pl.when(i == 0)
        def _first():
            lat_ref[0, 0] = part

        @pl.when(i > 0)
        def _rest():
            lat_ref[0, 0] += part


def _vq_argmin(flat, codebook, e2):
    n, d = flat.shape
    k = codebook.shape[0]
    grid = (n // _TB, k // _KB)
    return pl.pallas_call(
        _vq_tc_kernel,
        grid=grid,
        in_specs=[
            pl.BlockSpec((_TB, d), lambda i, j: (i, 0)),
            pl.BlockSpec((_KB, d), lambda i, j: (j, 0)),
            pl.BlockSpec((1, _KB), lambda i, j: (0, j)),
        ],
        out_specs=[
            pl.BlockSpec((_TB, 1), lambda i, j: (i, 0)),
            pl.BlockSpec((1, 1), lambda i, j: (0, 0)),
        ],
        out_shape=[
            jax.ShapeDtypeStruct((n, 1), jnp.int32),
            jax.ShapeDtypeStruct((1, 1), jnp.float32),
        ],
        scratch_shapes=[
            pltpu.VMEM((_TB, 1), jnp.float32),
            pltpu.VMEM((_TB, 1), jnp.int32),
        ],
    )(flat, codebook, e2)


def kernel(x, params):
    p = params
    h = jax.nn.relu(_conv1d(x, p['enc_c1_w'], p['enc_c1_b'], stride=2, pad=1, circular=True))
    h = jax.nn.relu(_conv1d(h, p['enc_c2_w'], p['enc_c2_b'], stride=2, pad=1, circular=True))
    h = jax.nn.relu(_conv1d(h, p['enc_c3_w'], p['enc_c3_b'], stride=2, pad=1, circular=True))
    h = jax.nn.relu(_conv1d(h, p['enc_c4_w'], p['enc_c4_b'], stride=2, pad=1, circular=True))
    h = _conv1d(h, p['enc_cf_w'], p['enc_cf_b'], stride=1, pad=1, circular=True)
    h = _residual_stack(h, [(p['enc_r0_w1'], p['enc_r0_w2']), (p['enc_r1_w1'], p['enc_r1_w2'])])
    z = _conv1d(h, p['pre_vq_w'], p['pre_vq_b'])

    zp = jnp.transpose(z, (0, 2, 1))
    flat = zp.reshape(-1, _EMBEDDING_DIM)
    E = p['codebook']
    e2 = jnp.sum(E ** 2, axis=1)[None, :]

    idx2d, latent_sum = _vq_argmin(flat, E, e2)
    idx = idx2d[:, 0]

    n = flat.shape[0]
    e_latent_loss = latent_sum[0, 0] / (n * _EMBEDDING_DIM)
    loss = _COMMITMENT_COST * e_latent_loss

    quantized = jnp.take(E, idx, axis=0)
    counts = jnp.bincount(idx, length=_NUM_EMBEDDINGS).astype(jnp.float32)

    avg_probs = counts / n
    perplexity = jnp.exp(-jnp.sum(avg_probs * jnp.log(avg_probs + 1e-10)))

    qc = jnp.transpose(quantized.reshape(zp.shape), (0, 2, 1))
    d = _conv1d(qc, p['dec_init_w'], p['dec_init_b'], stride=1, pad=1, circular=False)
    d = _residual_stack(d, [(p['dec_r0_w1'], p['dec_r0_w2']), (p['dec_r1_w1'], p['dec_r1_w2'])])
    d = jax.nn.relu(_conv_transpose1d(d, p['dec_t0_w'], p['dec_t0_b'], 2, 1))
    d = jax.nn.relu(_conv_transpose1d(d, p['dec_t1_w'], p['dec_t1_b'], 2, 1))
    d = jax.nn.relu(_conv_transpose1d(d, p['dec_t2_w'], p['dec_t2_b'], 2, 1))
    x_recon = _conv_transpose1d(d, p['dec_t3_w'], p['dec_t3_b'], 2, 1)
    return (loss, x_recon, perplexity)


# speedup vs baseline: 1.2297x; 1.2297x over previous
import functools

import jax
import jax.numpy as jnp
from jax import lax
from jax.experimental import pallas as pl
from jax.experimental.pallas import tpu as pltpu

_NUM_HIDDENS = 128
_NUM_EMBEDDINGS = 8192
_EMBEDDING_DIM = 64
_COMMITMENT_COST = 0.25

_TB = 1024
_KB = 1024


def _cpad(x, p):
    return jnp.concatenate([x[:, :, -p:], x, x[:, :, :p]], axis=2)


def _conv1d(x, W, b=None, stride=1, pad=0, circular=False):
    if circular and pad > 0:
        x = _cpad(x, pad)
        padding = ((0, 0),)
    else:
        padding = ((pad, pad),)
    out = lax.conv_general_dilated(x, W, (stride,), padding,
                                   dimension_numbers=('NCH', 'OIH', 'NCH'))
    if b is not None:
        out = out + b[None, :, None]
    return out


def _conv_transpose1d(x, W, b, stride, pad):
    k = W.shape[-1]
    out = lax.conv_general_dilated(x, W[:, :, ::-1], (1,),
                                   ((k - 1 - pad, k - 1 - pad),),
                                   lhs_dilation=(stride,),
                                   dimension_numbers=('NCH', 'OIH', 'NCH'))
    return out + b[None, :, None]


def _residual_stack(x, ws):
    for (w1, w2) in ws:
        h = jax.nn.relu(x)
        h = _conv1d(h, w1, stride=1, pad=1, circular=True)
        h = jax.nn.relu(h)
        h = _conv1d(h, w2)
        x = x + h
    return jax.nn.relu(x)


def _vq_tc_kernel(flat_ref, e_ref, e2_ref, idx_ref, lat_ref,
                  min_ref, arg_ref):
    j = pl.program_id(1)
    nk = pl.num_programs(1)
    i = pl.program_id(0)

    flat = flat_ref[...]
    e = e_ref[...]
    s = lax.dot_general(flat, e, (((1,), (1,)), ((), ())),
                        preferred_element_type=jnp.float32,
                        precision=lax.Precision.HIGHEST)
    s = e2_ref[...] - 2.0 * s

    m = jnp.min(s, axis=1, keepdims=True)
    col = lax.broadcasted_iota(jnp.int32, s.shape, 1)
    arg = jnp.min(jnp.where(s == m, col, _NUM_EMBEDDINGS), axis=1,
                  keepdims=True) + j * _KB

    @pl.when(j == 0)
    def _init():
        min_ref[...] = m
        arg_ref[...] = arg

    @pl.when(j > 0)
    def _update():
        better = m < min_ref[...]
        arg_ref[...] = jnp.where(better, arg, arg_ref[...])
        min_ref[...] = jnp.where(better, m, min_ref[...])

    @pl.when(j == nk - 1)
    def _finish():
        idx_ref[...] = arg_ref[...]
        z2 = jnp.sum(flat * flat, axis=1, keepdims=True)
        part = jnp.sum(min_ref[...] + z2).reshape(1, 1)

        @pl.when(i == 0)
        def _first():
            lat_ref[...] = part

        @pl.when(i > 0)
        def _rest():
            lat_ref[...] += part


def _vq_argmin(flat, codebook, e2):
    n, d = flat.shape
    k = codebook.shape[0]
    grid = (n // _TB, k // _KB)
    return pl.pallas_call(
        _vq_tc_kernel,
        grid=grid,
        in_specs=[
            pl.BlockSpec((_TB, d), lambda i, j: (i, 0)),
            pl.BlockSpec((_KB, d), lambda i, j: (j, 0)),
            pl.BlockSpec((1, _KB), lambda i, j: (0, j)),
        ],
        out_specs=[
            pl.BlockSpec((_TB, 1), lambda i, j: (i, 0)),
            pl.BlockSpec((1, 1), lambda i, j: (0, 0)),
        ],
        out_shape=[
            jax.ShapeDtypeStruct((n, 1), jnp.int32),
            jax.ShapeDtypeStruct((1, 1), jnp.float32),
        ],
        scratch_shapes=[
            pltpu.VMEM((_TB, 1), jnp.float32),
            pltpu.VMEM((_TB, 1), jnp.int32),
        ],
    )(flat, codebook, e2)


def kernel(x, params):
    p = params
    h = jax.nn.relu(_conv1d(x, p['enc_c1_w'], p['enc_c1_b'], stride=2, pad=1, circular=True))
    h = jax.nn.relu(_conv1d(h, p['enc_c2_w'], p['enc_c2_b'], stride=2, pad=1, circular=True))
    h = jax.nn.relu(_conv1d(h, p['enc_c3_w'], p['enc_c3_b'], stride=2, pad=1, circular=True))
    h = jax.nn.relu(_conv1d(h, p['enc_c4_w'], p['enc_c4_b'], stride=2, pad=1, circular=True))
    h = _conv1d(h, p['enc_cf_w'], p['enc_cf_b'], stride=1, pad=1, circular=True)
    h = _residual_stack(h, [(p['enc_r0_w1'], p['enc_r0_w2']), (p['enc_r1_w1'], p['enc_r1_w2'])])
    z = _conv1d(h, p['pre_vq_w'], p['pre_vq_b'])

    zp = jnp.transpose(z, (0, 2, 1))
    flat = zp.reshape(-1, _EMBEDDING_DIM)
    E = p['codebook']
    e2 = jnp.sum(E ** 2, axis=1)[None, :]

    idx2d, latent_sum = _vq_argmin(flat, E, e2)
    idx = idx2d[:, 0]

    n = flat.shape[0]
    e_latent_loss = latent_sum[0, 0] / (n * _EMBEDDING_DIM)
    loss = _COMMITMENT_COST * e_latent_loss

    quantized = jnp.take(E, idx, axis=0)
    counts = jnp.bincount(idx, length=_NUM_EMBEDDINGS).astype(jnp.float32)

    avg_probs = counts / n
    perplexity = jnp.exp(-jnp.sum(avg_probs * jnp.log(avg_probs + 1e-10)))

    qc = jnp.transpose(quantized.reshape(zp.shape), (0, 2, 1))
    d = _conv1d(qc, p['dec_init_w'], p['dec_init_b'], stride=1, pad=1, circular=False)
    d = _residual_stack(d, [(p['dec_r0_w1'], p['dec_r0_w2']), (p['dec_r1_w1'], p['dec_r1_w2'])])
    d = jax.nn.relu(_conv_transpose1d(d, p['dec_t0_w'], p['dec_t0_b'], 2, 1))
    d = jax.nn.relu(_conv_transpose1d(d, p['dec_t1_w'], p['dec_t1_b'], 2, 1))
    d = jax.nn.relu(_conv_transpose1d(d, p['dec_t2_w'], p['dec_t2_b'], 2, 1))
    x_recon = _conv_transpose1d(d, p['dec_t3_w'], p['dec_t3_b'], 2, 1)
    return (loss, x_recon, perplexity)


# speedup vs baseline: 1.7075x; 1.3886x over previous
import functools

import jax
import jax.numpy as jnp
from jax import lax
from jax.experimental import pallas as pl
from jax.experimental.pallas import tpu as pltpu

_NUM_HIDDENS = 128
_NUM_EMBEDDINGS = 8192
_EMBEDDING_DIM = 64
_COMMITMENT_COST = 0.25

_TB = 1024
_KB = 1024


def _cpad(x, p):
    return jnp.concatenate([x[:, :, -p:], x, x[:, :, :p]], axis=2)


def _conv1d(x, W, b=None, stride=1, pad=0, circular=False):
    if circular and pad > 0:
        x = _cpad(x, pad)
        padding = ((0, 0),)
    else:
        padding = ((pad, pad),)
    out = lax.conv_general_dilated(x, W, (stride,), padding,
                                   dimension_numbers=('NCH', 'OIH', 'NCH'))
    if b is not None:
        out = out + b[None, :, None]
    return out


def _conv_transpose1d(x, W, b, stride, pad):
    k = W.shape[-1]
    out = lax.conv_general_dilated(x, W[:, :, ::-1], (1,),
                                   ((k - 1 - pad, k - 1 - pad),),
                                   lhs_dilation=(stride,),
                                   dimension_numbers=('NCH', 'OIH', 'NCH'))
    return out + b[None, :, None]


def _residual_stack(x, ws):
    for (w1, w2) in ws:
        h = jax.nn.relu(x)
        h = _conv1d(h, w1, stride=1, pad=1, circular=True)
        h = jax.nn.relu(h)
        h = _conv1d(h, w2)
        x = x + h
    return jax.nn.relu(x)


def _vq_tc_kernel(flat_ref, et_ref, e2_ref, z2_ref, idx_ref, lat_ref,
                  min_ref, arg_ref):
    j = pl.program_id(1)
    nk = pl.num_programs(1)
    i = pl.program_id(0)

    flat = flat_ref[...]
    et = et_ref[...]
    m = lax.dot_general(flat, et, (((1,), (0,)), ((), ())),
                        preferred_element_type=jnp.float32,
                        precision=lax.Precision.DEFAULT)
    s = (z2_ref[...] + e2_ref[...]) - 2.0 * m

    m = jnp.min(s, axis=1, keepdims=True)
    col = lax.broadcasted_iota(jnp.int32, s.shape, 1)
    arg = jnp.min(jnp.where(s == m, col, _NUM_EMBEDDINGS), axis=1,
                  keepdims=True) + j * _KB

    @pl.when(j == 0)
    def _init():
        min_ref[...] = m
        arg_ref[...] = arg

    @pl.when(j > 0)
    def _update():
        better = m < min_ref[...]
        arg_ref[...] = jnp.where(better, arg, arg_ref[...])
        min_ref[...] = jnp.where(better, m, min_ref[...])

    @pl.when(j == nk - 1)
    def _finish():
        idx_ref[...] = arg_ref[...]
        part = jnp.sum(min_ref[...]).reshape(1, 1)

        @pl.when(i == 0)
        def _first():
            lat_ref[...] = part

        @pl.when(i > 0)
        def _rest():
            lat_ref[...] += part


def _vq_argmin(flat, et, e2, z2):
    n, d = flat.shape
    k = et.shape[1]
    grid = (n // _TB, k // _KB)
    return pl.pallas_call(
        _vq_tc_kernel,
        grid=grid,
        in_specs=[
            pl.BlockSpec((_TB, d), lambda i, j: (i, 0)),
            pl.BlockSpec((d, _KB), lambda i, j: (0, j)),
            pl.BlockSpec((1, _KB), lambda i, j: (0, j)),
            pl.BlockSpec((_TB, 1), lambda i, j: (i, 0)),
        ],
        out_specs=[
            pl.BlockSpec((_TB, 1), lambda i, j: (i, 0)),
            pl.BlockSpec((1, 1), lambda i, j: (0, 0)),
        ],
        out_shape=[
            jax.ShapeDtypeStruct((n, 1), jnp.int32),
            jax.ShapeDtypeStruct((1, 1), jnp.float32),
        ],
        scratch_shapes=[
            pltpu.VMEM((_TB, 1), jnp.float32),
            pltpu.VMEM((_TB, 1), jnp.int32),
        ],
    )(flat, et, e2, z2)


def kernel(x, params):
    p = params
    h = jax.nn.relu(_conv1d(x, p['enc_c1_w'], p['enc_c1_b'], stride=2, pad=1, circular=True))
    h = jax.nn.relu(_conv1d(h, p['enc_c2_w'], p['enc_c2_b'], stride=2, pad=1, circular=True))
    h = jax.nn.relu(_conv1d(h, p['enc_c3_w'], p['enc_c3_b'], stride=2, pad=1, circular=True))
    h = jax.nn.relu(_conv1d(h, p['enc_c4_w'], p['enc_c4_b'], stride=2, pad=1, circular=True))
    h = _conv1d(h, p['enc_cf_w'], p['enc_cf_b'], stride=1, pad=1, circular=True)
    h = _residual_stack(h, [(p['enc_r0_w1'], p['enc_r0_w2']), (p['enc_r1_w1'], p['enc_r1_w2'])])
    z = _conv1d(h, p['pre_vq_w'], p['pre_vq_b'])

    zp = jnp.transpose(z, (0, 2, 1))
    flat = zp.reshape(-1, _EMBEDDING_DIM)
    E = p['codebook']
    e2 = jnp.sum(E ** 2, axis=1)[None, :]
    z2 = jnp.sum(flat ** 2, axis=1, keepdims=True)

    idx2d, latent_sum = _vq_argmin(flat, E.T, e2, z2)
    idx = idx2d[:, 0]

    n = flat.shape[0]
    e_latent_loss = latent_sum[0, 0] / (n * _EMBEDDING_DIM)
    loss = _COMMITMENT_COST * e_latent_loss

    quantized = jnp.take(E, idx, axis=0)
    counts = jnp.bincount(idx, length=_NUM_EMBEDDINGS).astype(jnp.float32)

    avg_probs = counts / n
    perplexity = jnp.exp(-jnp.sum(avg_probs * jnp.log(avg_probs + 1e-10)))

    qc = jnp.transpose(quantized.reshape(zp.shape), (0, 2, 1))
    d = _conv1d(qc, p['dec_init_w'], p['dec_init_b'], stride=1, pad=1, circular=False)
    d = _residual_stack(d, [(p['dec_r0_w1'], p['dec_r0_w2']), (p['dec_r1_w1'], p['dec_r1_w2'])])
    d = jax.nn.relu(_conv_transpose1d(d, p['dec_t0_w'], p['dec_t0_b'], 2, 1))
    d = jax.nn.relu(_conv_transpose1d(d, p['dec_t1_w'], p['dec_t1_b'], 2, 1))
    d = jax.nn.relu(_conv_transpose1d(d, p['dec_t2_w'], p['dec_t2_b'], 2, 1))
    x_recon = _conv_transpose1d(d, p['dec_t3_w'], p['dec_t3_b'], 2, 1)
    return (loss, x_recon, perplexity)
